# Initial kernel scaffold; baseline (speedup 1.0000x reference)
import jax
import jax.numpy as jnp
from jax import lax
from jax.experimental import pallas as pl
from jax.experimental.pallas import tpu as pltpu
from jax.experimental.pallas import tpu_sc as plsc

VOCAB = 1000000
DIM = 32
BATCH = 16384
HIST = 200

_NC = 2
_NS = 16
_NW = _NC * _NS

_B_FLAT = BATCH * HIST
_PER_W = _B_FLAT // _NW
_GATHER = 128
_K = 8
_CHUNK = _K * _GATHER
_NCHUNK = _PER_W // _CHUNK


def _body(table_hbm, idx_hbm, out_hbm, idx_v, rows_v, sem):
    wid = lax.axis_index("s") * _NC + lax.axis_index("c")

    @pl.loop(0, _NCHUNK)
    def _chunk(c):
        pltpu.sync_copy(idx_hbm.at[wid, pl.ds(c * _K, _K)], idx_v)
        for j in range(_K):
            pltpu.async_copy(
                table_hbm.at[idx_v.at[j]],
                rows_v.at[pl.ds(j * _GATHER, _GATHER)],
                sem,
            )
        for j in range(_K):
            pltpu.make_async_copy(
                table_hbm.at[idx_v.at[j]],
                rows_v.at[pl.ds(j * _GATHER, _GATHER)],
                sem,
            ).wait()
        pltpu.sync_copy(
            rows_v, out_hbm.at[pl.ds(wid * _PER_W + c * _CHUNK, _CHUNK)]
        )


@jax.jit
def _embed(x_grouped, embedding):
    k = pl.kernel(
        _body,
        out_type=jax.ShapeDtypeStruct((_B_FLAT, DIM), jnp.float32),
        mesh=plsc.VectorSubcoreMesh(
            core_axis_name="c", subcore_axis_name="s", num_cores=_NC
        ),
        scratch_types=[
            pltpu.VMEM((_K, _GATHER), jnp.int32),
            pltpu.VMEM((_CHUNK, DIM), jnp.float32),
            pltpu.SemaphoreType.DMA,
        ],
    )
    return k(embedding, x_grouped)


def kernel(x, embedding):
    x_grouped = x.reshape(_NW, _PER_W // _GATHER, _GATHER)
    out = _embed(x_grouped, embedding)
    return out.reshape(BATCH, HIST, DIM)


# speedup vs baseline: 4.8082x; 4.8082x over previous
import jax
import jax.numpy as jnp
from jax import lax
from jax.experimental import pallas as pl
from jax.experimental.pallas import tpu as pltpu
from jax.experimental.pallas import tpu_sc as plsc

VOCAB = 1000000
DIM = 32
BATCH = 16384
HIST = 200

_NC = 2
_NS = 16
_NW = _NC * _NS

_B_FLAT = BATCH * HIST
_PER_W = _B_FLAT // _NW
_GATHER = 128
_K = 8
_CHUNK = _K * _GATHER
_NCHUNK = _PER_W // _CHUNK


def _body(table_hbm, idx_hbm, out_hbm, idx_v, rows_v, sem):
    wid = lax.axis_index("s") * _NC + lax.axis_index("c")

    @pl.loop(0, _NCHUNK)
    def _chunk(c):
        pltpu.sync_copy(idx_hbm.at[wid, pl.ds(c * _K, _K)], idx_v)
        for j in range(_K):
            pltpu.async_copy(
                table_hbm.at[idx_v.at[j]],
                rows_v.at[pl.ds(j * _GATHER, _GATHER)],
                sem,
            )
        for j in range(_K):
            pltpu.make_async_copy(
                table_hbm.at[idx_v.at[j]],
                rows_v.at[pl.ds(j * _GATHER, _GATHER)],
                sem,
            ).wait()
        pltpu.sync_copy(
            rows_v, out_hbm.at[pl.ds(wid * _PER_W + c * _CHUNK, _CHUNK)]
        )


@jax.jit
def _embed(x_grouped, embedding):
    k = pl.kernel(
        _body,
        out_type=jax.ShapeDtypeStruct((_B_FLAT, DIM), jnp.float32),
        mesh=plsc.VectorSubcoreMesh(
            core_axis_name="c", subcore_axis_name="s", num_cores=_NC
        ),
        scratch_types=[
            pltpu.VMEM((_K, _GATHER), jnp.int32),
            pltpu.VMEM((_CHUNK, DIM), jnp.float32),
            pltpu.SemaphoreType.DMA,
        ],
        compiler_params=pltpu.CompilerParams(use_tc_tiling_on_sc=False),
    )
    return k(embedding, x_grouped)


def kernel(x, embedding):
    x_grouped = x.reshape(_NW, _PER_W // _GATHER, _GATHER)
    out = _embed(x_grouped, embedding)
    return out.reshape(BATCH, HIST, DIM)


# speedup vs baseline: 5.0323x; 1.0466x over previous
import jax
import jax.numpy as jnp
from jax import lax
from jax.experimental import pallas as pl
from jax.experimental.pallas import tpu as pltpu
from jax.experimental.pallas import tpu_sc as plsc

VOCAB = 1000000
DIM = 32
BATCH = 16384
HIST = 200

_NC = 2
_NS = 16
_NW = _NC * _NS

_B_FLAT = BATCH * HIST
_PER_W = _B_FLAT // _NW
_GATHER = 128
_K = 8
_CHUNK = _K * _GATHER
_NCHUNK = _PER_W // _CHUNK
_NGRP = _NCHUNK // 2


def _body(table_hbm, idx_hbm, out_hbm,
          idx0, idx1, rows0, rows1,
          sem_i0, sem_i1, sem_g0, sem_g1, sem_o0, sem_o1):
    wid = lax.axis_index("s") * _NC + lax.axis_index("c")
    idx_v = [idx0, idx1]
    rows_v = [rows0, rows1]
    sem_i = [sem_i0, sem_i1]
    sem_g = [sem_g0, sem_g1]
    sem_o = [sem_o0, sem_o1]

    def idx_copy(c, b):
        return pltpu.make_async_copy(
            idx_hbm.at[wid, pl.ds(c * _K, _K)], idx_v[b], sem_i[b]
        )

    def gather_copy(j, b):
        return pltpu.make_async_copy(
            table_hbm.at[idx_v[b].at[j]],
            rows_v[b].at[pl.ds(j * _GATHER, _GATHER)],
            sem_g[b],
        )

    def out_copy(c, b):
        return pltpu.make_async_copy(
            rows_v[b],
            out_hbm.at[pl.ds(wid * _PER_W + c * _CHUNK, _CHUNK)],
            sem_o[b],
        )

    def gather_chunk(b):
        for j in range(_K):
            gather_copy(j, b).start()
        for j in range(_K):
            gather_copy(j, b).wait()

    idx_copy(0, 0).start()
    idx_copy(1, 1).start()
    idx_copy(0, 0).wait()
    gather_chunk(0)
    out_copy(0, 0).start()
    idx_copy(2, 0).start()
    idx_copy(1, 1).wait()
    gather_chunk(1)
    out_copy(1, 1).start()

    @pl.loop(1, _NGRP - 1)
    def _grp(g):
        for b in range(2):
            c = 2 * g + b
            idx_copy(c + 1, 1 - b).start()
            idx_copy(c, b).wait()
            out_copy(c - 2, b).wait()
            gather_chunk(b)
            out_copy(c, b).start()

    c0 = _NCHUNK - 2
    idx_copy(c0 + 1, 1).start()
    idx_copy(c0, 0).wait()
    out_copy(c0 - 2, 0).wait()
    gather_chunk(0)
    out_copy(c0, 0).start()
    idx_copy(c0 + 1, 1).wait()
    out_copy(c0 - 1, 1).wait()
    gather_chunk(1)
    out_copy(c0 + 1, 1).start()
    out_copy(c0, 0).wait()
    out_copy(c0 + 1, 1).wait()


@jax.jit
def _embed(x_grouped, embedding):
    k = pl.kernel(
        _body,
        out_type=jax.ShapeDtypeStruct((_B_FLAT, DIM), jnp.float32),
        mesh=plsc.VectorSubcoreMesh(
            core_axis_name="c", subcore_axis_name="s", num_cores=_NC
        ),
        scratch_types=[
            pltpu.VMEM((_K, _GATHER), jnp.int32),
            pltpu.VMEM((_K, _GATHER), jnp.int32),
            pltpu.VMEM((_CHUNK, DIM), jnp.float32),
            pltpu.VMEM((_CHUNK, DIM), jnp.float32),
            pltpu.SemaphoreType.DMA,
            pltpu.SemaphoreType.DMA,
            pltpu.SemaphoreType.DMA,
            pltpu.SemaphoreType.DMA,
            pltpu.SemaphoreType.DMA,
            pltpu.SemaphoreType.DMA,
        ],
        compiler_params=pltpu.CompilerParams(use_tc_tiling_on_sc=False),
    )
    return k(embedding, x_grouped)


def kernel(x, embedding):
    x_grouped = x.reshape(_NW, _PER_W // _GATHER, _GATHER)
    out = _embed(x_grouped, embedding)
    return out.reshape(BATCH, HIST, DIM)
